# baseline (speedup 1.0000x reference)
import functools

import jax
import jax.numpy as jnp
from jax import lax
from jax.experimental import pallas as pl
from jax.experimental.pallas import tpu as pltpu
from jax.experimental.pallas import tpu_sc as plsc

L = 256
HW = 512 * 1024
R = 6 * L * L
NW = 32
CHUNK = HW // NW
S = 256
NSUB = CHUNK // S
IB = 128
NB = S // IB


def _body(xt_h, geom_h, oiz_h, out_h,
          oi_v, geomA, geomB, rA, rB, wA, wB, gA, gB, outA, outB,
          isem, gsemA, gsemB, ssemA, ssemB):
    cid = lax.axis_index("c")
    sid = lax.axis_index("s")
    wid = sid * 2 + cid
    base = wid * CHUNK
    mbase = wid * NSUB

    geom = [geomA, geomB]
    rv = [rA, rB]
    wv = [wA, wB]
    gv = [gA, gB]
    ov = [outA, outB]
    gsem = [gsemA, gsemB]
    ssem = [ssemA, ssemB]

    def drain(dst, sem, src):
        pltpu.make_async_copy(src, dst, sem).wait()

    def addr(p):
        def one(q, c2):
            cs = pl.ds(q * 16, 16)
            gxv = geom[p][0, cs]
            gyv = geom[p][1, cs]
            fbv = geom[p][2, cs].astype(jnp.int32)
            x0 = gxv.astype(jnp.int32)
            y0 = gyv.astype(jnp.int32)
            wx = gxv - x0.astype(jnp.float32)
            wy = gyv - y0.astype(jnp.float32)
            y1 = jnp.minimum(y0 + 1, L - 1)
            r0 = fbv + y0 * L + x0
            r1 = fbv + y1 * L + x0
            rv[p][0, cs] = r0
            rv[p][1, cs] = jnp.minimum(r0 + 1, R - 1)
            rv[p][2, cs] = r1
            rv[p][3, cs] = jnp.minimum(r1 + 1, R - 1)
            wv[p][0, cs] = (1.0 - wx) * (1.0 - wy)
            wv[p][1, cs] = wx * (1.0 - wy)
            wv[p][2, cs] = (1.0 - wx) * wy
            wv[p][3, cs] = wx * wy
            return c2
        lax.fori_loop(0, S // 16, one, 0)

    def fire_gathers(p):
        for tap in range(4):
            for j in range(NB):
                ds = pl.ds(j * IB, IB)
                pltpu.async_copy(xt_h.at[rv[p].at[tap, ds]],
                                 gv[p].at[tap, ds], gsem[p])

    def drain_gathers(p):
        for tap in range(4):
            drain(gv[p].at[tap], gsem[p], xt_h.at[pl.ds(0, S)])

    def combine(p):
        def one(g, c2):
            sl = pl.ds(g * 16, 16)
            wa = wv[p][0, sl]
            wb = wv[p][1, sl]
            wc = wv[p][2, sl]
            wd = wv[p][3, sl]
            lo = pl.ds(0, 16)
            hi = pl.ds(16, 16)
            for k in range(16):
                e = g * 16 + k
                ov[p][e, lo] = (gv[p][0, e, lo] * wa[k] + gv[p][1, e, lo] * wb[k]
                                + gv[p][2, e, lo] * wc[k] + gv[p][3, e, lo] * wd[k])
                ov[p][e, hi] = (gv[p][0, e, hi] * wa[k] + gv[p][1, e, hi] * wb[k]
                                + gv[p][2, e, hi] * wc[k] + gv[p][3, e, hi] * wd[k])
            return c2
        lax.fori_loop(0, S // 16, one, 0)

    def phase(t, par, drain_scatter):
        nxt = 1 - par
        drain(geom[nxt], isem, geom_h.at[0])
        addr(nxt)
        fire_gathers(nxt)
        pltpu.async_copy(geom_h.at[mbase + t + 2], geom[par], isem)
        if drain_scatter:
            drain(ov[par], ssem[par], xt_h.at[pl.ds(0, S)])
        drain_gathers(par)
        combine(par)
        for j in range(NB):
            pltpu.async_copy(ov[par].at[pl.ds(j * IB, IB)],
                             out_h.at[oi_v.at[t * NB + j]], ssem[par])

    pltpu.sync_copy(oiz_h.at[pl.ds(wid * (CHUNK // IB), CHUNK // IB)], oi_v)
    pltpu.sync_copy(geom_h.at[mbase], geomA)
    addr(0)
    fire_gathers(0)
    pltpu.async_copy(geom_h.at[mbase + 1], geomB, isem)
    phase(0, 0, False)
    phase(1, 1, False)

    def step(k, c2):
        t = 2 * k + 2
        phase(t, 0, True)
        phase(t + 1, 1, True)
        return c2
    lax.fori_loop(0, (NSUB - 2) // 2, step, 0)

    drain_gathers(0)
    drain(geom[1], isem, geom_h.at[0])
    drain(ov[0], ssem[0], xt_h.at[pl.ds(0, S)])
    drain(ov[1], ssem[1], xt_h.at[pl.ds(0, S)])


@jax.jit
def _run(xt, geom, oiz):
    mesh = plsc.VectorSubcoreMesh(core_axis_name="c", subcore_axis_name="s")
    f = functools.partial(
        pl.kernel,
        out_type=jax.ShapeDtypeStruct((HW, 32), jnp.float32),
        mesh=mesh,
        compiler_params=pltpu.CompilerParams(use_tc_tiling_on_sc=False),
        scratch_types=[
            pltpu.VMEM((CHUNK // IB, IB), jnp.int32),
            pltpu.VMEM((3, S), jnp.float32),
            pltpu.VMEM((3, S), jnp.float32),
            pltpu.VMEM((4, S), jnp.int32),
            pltpu.VMEM((4, S), jnp.int32),
            pltpu.VMEM((4, S), jnp.float32),
            pltpu.VMEM((4, S), jnp.float32),
            pltpu.VMEM((4, S, 32), jnp.float32),
            pltpu.VMEM((4, S, 32), jnp.float32),
            pltpu.VMEM((S, 32), jnp.float32),
            pltpu.VMEM((S, 32), jnp.float32),
            pltpu.SemaphoreType.DMA,
            pltpu.SemaphoreType.DMA,
            pltpu.SemaphoreType.DMA,
            pltpu.SemaphoreType.DMA,
            pltpu.SemaphoreType.DMA,
        ],
    )(_body)
    return f(xt, geom, oiz)


_BP = 8192
_BO = 16384


def _repack_body(x_ref, o_ref):
    blk = x_ref[:, 0, :, :].reshape(32, _BP)
    o_ref[0] = jnp.transpose(blk, (1, 0))


def _outt_body(i_ref, o_ref):
    for k in range(_BO // 128):
        o_ref[:, k, :] = jnp.transpose(i_ref[pl.ds(k * 128, 128), :], (1, 0))


@jax.jit
def _repack(x):
    xv = x.reshape(2, 6, 16, L * L)
    out = pl.pallas_call(
        _repack_body,
        grid=(6, (L * L) // _BP),
        in_specs=[pl.BlockSpec((2, 1, 16, _BP), lambda i, j: (0, i, 0, j))],
        out_specs=pl.BlockSpec((1, _BP, 32), lambda i, j: (i, j, 0)),
        out_shape=jax.ShapeDtypeStruct((6, L * L, 32), jnp.float32),
    )(xv)
    return out.reshape(R, 32)


@jax.jit
def _outt(out_pm):
    out = pl.pallas_call(
        _outt_body,
        grid=(HW // _BO,),
        in_specs=[pl.BlockSpec((_BO, 32), lambda j: (j, 0))],
        out_specs=pl.BlockSpec((32, _BO // 128, 128), lambda j: (0, j, 0)),
        out_shape=jax.ShapeDtypeStruct((32, HW // 128, 128), jnp.float32),
    )(out_pm)
    return out.reshape(32, HW)


def kernel(x, XY0, idx0, XY1, idx1, XY2, idx2, XY3, idx3, XY4, idx4, XY5, idx5):
    XYs = [XY0, XY1, XY2, XY3, XY4, XY5]
    idxs = [idx0, idx1, idx2, idx3, idx4, idx5]
    equ = x.shape[0] // 6
    C = x.shape[1]
    xt = _repack(x)
    gx = jnp.concatenate([xy[:, 0] for xy in XYs])
    gy = jnp.concatenate([xy[:, 1] for xy in XYs])
    fb = jnp.concatenate([jnp.full((idxs[f].shape[0],), f * L * L, jnp.int32)
                          for f in range(6)])
    oi = jnp.concatenate(idxs)
    geom = jnp.stack([gx, gy, fb.astype(jnp.float32)])
    geom = geom.reshape(3, HW // S, S).transpose(1, 0, 2)
    geom = jnp.pad(geom, ((0, 2), (0, 0), (0, 0)))
    oiz = oi.reshape(HW // IB, IB)
    out_pm = _run(xt, geom, oiz)
    return _outt(out_pm).reshape(equ, C, 512, 1024)


# speedup vs baseline: 1.0032x; 1.0032x over previous
import functools

import jax
import jax.numpy as jnp
from jax import lax
from jax.experimental import pallas as pl
from jax.experimental.pallas import tpu as pltpu
from jax.experimental.pallas import tpu_sc as plsc

L = 256
HW = 512 * 1024
R = 6 * L * L
NW = 32
CHUNK = HW // NW
S = 128
NSUB = CHUNK // S
GP = 4


def _body(xt_h, geom_h, oiz_h, out_h,
          oi_v, geom_v, r_v, w_v, g_v, outA, outB,
          isemA, isemB, gsem0, gsem1, gsem2, gsem3, ssemA, ssemB):
    cid = lax.axis_index("c")
    sid = lax.axis_index("s")
    wid = sid * 2 + cid
    base = wid * CHUNK
    mbase = wid * NSUB

    ov = [outA, outB]
    isem = [isemA, isemB]
    gsem = [gsem0, gsem1, gsem2, gsem3]
    ssem = [ssemA, ssemB]

    def drain(dst, sem, src):
        pltpu.make_async_copy(src, dst, sem).wait()

    def addr(t, q4):
        def one(q, c2):
            cs = pl.ds(q * 16, 16)
            gxv = geom_v[q4, 0, cs]
            gyv = geom_v[q4, 1, cs]
            fbv = geom_v[q4, 2, cs].astype(jnp.int32)
            x0 = gxv.astype(jnp.int32)
            y0 = gyv.astype(jnp.int32)
            wx = gxv - x0.astype(jnp.float32)
            wy = gyv - y0.astype(jnp.float32)
            y1 = jnp.minimum(y0 + 1, L - 1)
            r0 = fbv + y0 * L + x0
            r1 = fbv + y1 * L + x0
            r_v[q4, 0, cs] = r0
            r_v[q4, 1, cs] = jnp.minimum(r0 + 1, R - 1)
            r_v[q4, 2, cs] = r1
            r_v[q4, 3, cs] = jnp.minimum(r1 + 1, R - 1)
            w_v[q4, 0, cs] = (1.0 - wx) * (1.0 - wy)
            w_v[q4, 1, cs] = wx * (1.0 - wy)
            w_v[q4, 2, cs] = (1.0 - wx) * wy
            w_v[q4, 3, cs] = wx * wy
            return c2
        lax.fori_loop(0, S // 16, one, 0)

    def fire_gathers(q4):
        for tap in range(4):
            pltpu.async_copy(xt_h.at[r_v.at[q4, tap]],
                             g_v.at[q4, tap], gsem[q4])

    def drain_gathers(q4):
        for tap in range(4):
            drain(g_v.at[q4, tap], gsem[q4], xt_h.at[pl.ds(0, S)])

    def combine(q4, p2):
        def one(g, c2):
            sl = pl.ds(g * 16, 16)
            wa = w_v[q4, 0, sl]
            wb = w_v[q4, 1, sl]
            wc = w_v[q4, 2, sl]
            wd = w_v[q4, 3, sl]
            lo = pl.ds(0, 16)
            hi = pl.ds(16, 16)
            for k in range(16):
                e = g * 16 + k
                ov[p2][e, lo] = (g_v[q4, 0, e, lo] * wa[k] + g_v[q4, 1, e, lo] * wb[k]
                                 + g_v[q4, 2, e, lo] * wc[k] + g_v[q4, 3, e, lo] * wd[k])
                ov[p2][e, hi] = (g_v[q4, 0, e, hi] * wa[k] + g_v[q4, 1, e, hi] * wb[k]
                                 + g_v[q4, 2, e, hi] * wc[k] + g_v[q4, 3, e, hi] * wd[k])
            return c2
        lax.fori_loop(0, S // 16, one, 0)

    def phase(t, q4, p2, drain_scatter):
        q4n2 = (q4 + 2) % 4
        drain(geom_v.at[q4n2], isem[p2], geom_h.at[0])
        addr(t + 2, q4n2)
        fire_gathers(q4n2)
        pltpu.async_copy(geom_h.at[mbase + t + 4], geom_v.at[q4], isem[p2])
        drain_gathers(q4)
        if drain_scatter:
            drain(ov[p2], ssem[p2], xt_h.at[pl.ds(0, S)])
        combine(q4, p2)
        pltpu.async_copy(ov[p2], out_h.at[oi_v.at[t]], ssem[p2])

    pltpu.sync_copy(oiz_h.at[pl.ds(wid * NSUB, NSUB)], oi_v)
    pltpu.sync_copy(geom_h.at[mbase], geom_v.at[0])
    pltpu.sync_copy(geom_h.at[mbase + 1], geom_v.at[1])
    addr(0, 0)
    addr(1, 1)
    fire_gathers(0)
    fire_gathers(1)
    pltpu.async_copy(geom_h.at[mbase + 2], geom_v.at[2], isem[0])
    pltpu.async_copy(geom_h.at[mbase + 3], geom_v.at[3], isem[1])
    phase(0, 0, 0, False)
    phase(1, 1, 1, False)
    phase(2, 2, 0, True)
    phase(3, 3, 1, True)

    def step(k, c2):
        t = 4 * k
        phase(t, 0,0, True)
        phase(t + 1, 1, 1, True)
        phase(t + 2, 2, 0, True)
        phase(t + 3, 3, 1, True)
        return c2
    lax.fori_loop(1, NSUB // 4, step, 0)

    drain_gathers(0)
    drain_gathers(1)
    drain(geom_v.at[2], isem[0], geom_h.at[0])
    drain(geom_v.at[3], isem[1], geom_h.at[0])
    drain(ov[0], ssem[0], xt_h.at[pl.ds(0, S)])
    drain(ov[1], ssem[1], xt_h.at[pl.ds(0, S)])


@jax.jit
def _run(xt, geom, oiz):
    mesh = plsc.VectorSubcoreMesh(core_axis_name="c", subcore_axis_name="s")
    f = functools.partial(
        pl.kernel,
        out_type=jax.ShapeDtypeStruct((HW, 32), jnp.float32),
        mesh=mesh,
        compiler_params=pltpu.CompilerParams(use_tc_tiling_on_sc=False),
        scratch_types=[
            pltpu.VMEM((NSUB, S), jnp.int32),
            pltpu.VMEM((GP, 3, S), jnp.float32),
            pltpu.VMEM((GP, 4, S), jnp.int32),
            pltpu.VMEM((GP, 4, S), jnp.float32),
            pltpu.VMEM((GP, 4, S, 32), jnp.float32),
            pltpu.VMEM((S, 32), jnp.float32),
            pltpu.VMEM((S, 32), jnp.float32),
            pltpu.SemaphoreType.DMA,
            pltpu.SemaphoreType.DMA,
            pltpu.SemaphoreType.DMA,
            pltpu.SemaphoreType.DMA,
            pltpu.SemaphoreType.DMA,
            pltpu.SemaphoreType.DMA,
            pltpu.SemaphoreType.DMA,
            pltpu.SemaphoreType.DMA,
        ],
    )(_body)
    return f(xt, geom, oiz)


_BP = 8192
_BO = 16384


def _repack_body(x_ref, o_ref):
    blk = x_ref[:, 0, :, :].reshape(32, _BP)
    o_ref[0] = jnp.transpose(blk, (1, 0))


def _outt_body(i_ref, o_ref):
    for k in range(_BO // 128):
        o_ref[:, k, :] = jnp.transpose(i_ref[pl.ds(k * 128, 128), :], (1, 0))


@jax.jit
def _repack(x):
    xv = x.reshape(2, 6, 16, L * L)
    out = pl.pallas_call(
        _repack_body,
        grid=(6, (L * L) // _BP),
        in_specs=[pl.BlockSpec((2, 1, 16, _BP), lambda i, j: (0, i, 0, j))],
        out_specs=pl.BlockSpec((1, _BP, 32), lambda i, j: (i, j, 0)),
        out_shape=jax.ShapeDtypeStruct((6, L * L, 32), jnp.float32),
    )(xv)
    return out.reshape(R, 32)


@jax.jit
def _outt(out_pm):
    out = pl.pallas_call(
        _outt_body,
        grid=(HW // _BO,),
        in_specs=[pl.BlockSpec((_BO, 32), lambda j: (j, 0))],
        out_specs=pl.BlockSpec((32, _BO // 128, 128), lambda j: (0, j, 0)),
        out_shape=jax.ShapeDtypeStruct((32, HW // 128, 128), jnp.float32),
    )(out_pm)
    return out.reshape(32, HW)


def kernel(x, XY0, idx0, XY1, idx1, XY2, idx2, XY3, idx3, XY4, idx4, XY5, idx5):
    XYs = [XY0, XY1, XY2, XY3, XY4, XY5]
    idxs = [idx0, idx1, idx2, idx3, idx4, idx5]
    equ = x.shape[0] // 6
    C = x.shape[1]
    xt = _repack(x)
    gx = jnp.concatenate([xy[:, 0] for xy in XYs])
    gy = jnp.concatenate([xy[:, 1] for xy in XYs])
    fb = jnp.concatenate([jnp.full((idxs[f].shape[0],), f * L * L, jnp.int32)
                          for f in range(6)])
    oi = jnp.concatenate(idxs)
    geom = jnp.stack([gx, gy, fb.astype(jnp.float32)])
    geom = geom.reshape(3, HW // S, S).transpose(1, 0, 2)
    geom = jnp.pad(geom, ((0, 4), (0, 0), (0, 0)))
    oiz = oi.reshape(HW // S, S)
    out_pm = _run(xt, geom, oiz)
    return _outt(out_pm).reshape(equ, C, 512, 1024)


# speedup vs baseline: 1.0581x; 1.0548x over previous
import functools

import jax
import jax.numpy as jnp
from jax import lax
from jax.experimental import pallas as pl
from jax.experimental.pallas import tpu as pltpu
from jax.experimental.pallas import tpu_sc as plsc

L = 256
HW = 512 * 1024
R = 6 * L * L
NW = 32
CHUNK = HW // NW
S = 128
NSUB = CHUNK // S
GP = 4


def _body(xt_h, geom_h, oiz_h, out_h,
          oi_v, geom_v, r_v, w_v, g_v, outA, outB,
          isemA, isemB, gsem0, gsem1, gsem2, gsem3, ssemA, ssemB):
    cid = lax.axis_index("c")
    sid = lax.axis_index("s")
    wid = sid * 2 + cid
    base = wid * CHUNK
    mbase = wid * NSUB

    ov = [outA, outB]
    isem = [isemA, isemB]
    gsem = [gsem0, gsem1, gsem2, gsem3]
    ssem = [ssemA, ssemB]

    def drain(dst, sem, src):
        pltpu.make_async_copy(src, dst, sem).wait()

    def addr(t, q4):
        def one(q, c2):
            cs = pl.ds(q * 16, 16)
            gxv = geom_v[q4, 0, cs]
            gyv = geom_v[q4, 1, cs]
            fbv = geom_v[q4, 2, cs].astype(jnp.int32)
            x0 = gxv.astype(jnp.int32)
            y0 = gyv.astype(jnp.int32)
            wx = gxv - x0.astype(jnp.float32)
            wy = gyv - y0.astype(jnp.float32)
            y1 = jnp.minimum(y0 + 1, L - 1)
            r0 = fbv + y0 * L + x0
            r1 = fbv + y1 * L + x0
            r_v[q4, 0, cs] = r0
            r_v[q4, 1, cs] = jnp.minimum(r0 + 1, R - 1)
            r_v[q4, 2, cs] = r1
            r_v[q4, 3, cs] = jnp.minimum(r1 + 1, R - 1)
            w_v[q4, 0, cs] = (1.0 - wx) * (1.0 - wy)
            w_v[q4, 1, cs] = wx * (1.0 - wy)
            w_v[q4, 2, cs] = (1.0 - wx) * wy
            w_v[q4, 3, cs] = wx * wy
            return c2
        lax.fori_loop(0, S // 16, one, 0)

    def fire_gathers(q4):
        for tap in range(4):
            pltpu.async_copy(xt_h.at[r_v.at[q4, tap]],
                             g_v.at[q4, tap], gsem[q4])

    def drain_gathers(q4):
        for tap in range(4):
            drain(g_v.at[q4, tap], gsem[q4], xt_h.at[pl.ds(0, S)])

    def combine(q4, p2):
        def one(g, c2):
            sl = pl.ds(g * 16, 16)
            wa = w_v[q4, 0, sl]
            wb = w_v[q4, 1, sl]
            wc = w_v[q4, 2, sl]
            wd = w_v[q4, 3, sl]
            lo = pl.ds(0, 16)
            hi = pl.ds(16, 16)
            for k in range(16):
                e = g * 16 + k
                ov[p2][e, lo] = (g_v[q4, 0, e, lo] * wa[k] + g_v[q4, 1, e, lo] * wb[k]
                                 + g_v[q4, 2, e, lo] * wc[k] + g_v[q4, 3, e, lo] * wd[k])
                ov[p2][e, hi] = (g_v[q4, 0, e, hi] * wa[k] + g_v[q4, 1, e, hi] * wb[k]
                                 + g_v[q4, 2, e, hi] * wc[k] + g_v[q4, 3, e, hi] * wd[k])
            return c2
        lax.fori_loop(0, S // 16, one, 0)

    def phase(t, q4, p2, drain_scatter):
        q4n2 = (q4 + 2) % 4
        drain(geom_v.at[q4n2], isem[p2], geom_h.at[0])
        addr(t + 2, q4n2)
        fire_gathers(q4n2)
        pltpu.async_copy(geom_h.at[mbase + t + 4], geom_v.at[q4], isem[p2])
        drain_gathers(q4)
        if drain_scatter:
            drain(ov[p2], ssem[p2], xt_h.at[pl.ds(0, S)])
        combine(q4, p2)
        pltpu.async_copy(ov[p2], out_h.at[oi_v.at[t]], ssem[p2])

    pltpu.sync_copy(oiz_h.at[pl.ds(wid * NSUB, NSUB)], oi_v)
    pltpu.sync_copy(geom_h.at[mbase], geom_v.at[0])
    pltpu.sync_copy(geom_h.at[mbase + 1], geom_v.at[1])
    addr(0, 0)
    addr(1, 1)
    fire_gathers(0)
    fire_gathers(1)
    pltpu.async_copy(geom_h.at[mbase + 2], geom_v.at[2], isem[0])
    pltpu.async_copy(geom_h.at[mbase + 3], geom_v.at[3], isem[1])
    phase(0, 0, 0, False)
    phase(1, 1, 1, False)
    phase(2, 2, 0, True)
    phase(3, 3, 1, True)

    def step(k, c2):
        t = 4 * k
        phase(t, 0,0, True)
        phase(t + 1, 1, 1, True)
        phase(t + 2, 2, 0, True)
        phase(t + 3, 3, 1, True)
        return c2
    lax.fori_loop(1, NSUB // 4, step, 0)

    drain_gathers(0)
    drain_gathers(1)
    drain(geom_v.at[2], isem[0], geom_h.at[0])
    drain(geom_v.at[3], isem[1], geom_h.at[0])
    drain(ov[0], ssem[0], xt_h.at[pl.ds(0, S)])
    drain(ov[1], ssem[1], xt_h.at[pl.ds(0, S)])


@jax.jit
def _run(xt, geom, oiz):
    mesh = plsc.VectorSubcoreMesh(core_axis_name="c", subcore_axis_name="s")
    f = functools.partial(
        pl.kernel,
        out_type=jax.ShapeDtypeStruct((HW, 32), jnp.float32),
        mesh=mesh,
        compiler_params=pltpu.CompilerParams(use_tc_tiling_on_sc=False),
        scratch_types=[
            pltpu.VMEM((NSUB, S), jnp.int32),
            pltpu.VMEM((GP, 3, S), jnp.float32),
            pltpu.VMEM((GP, 4, S), jnp.int32),
            pltpu.VMEM((GP, 4, S), jnp.float32),
            pltpu.VMEM((GP, 4, S, 32), jnp.float32),
            pltpu.VMEM((S, 32), jnp.float32),
            pltpu.VMEM((S, 32), jnp.float32),
            pltpu.SemaphoreType.DMA,
            pltpu.SemaphoreType.DMA,
            pltpu.SemaphoreType.DMA,
            pltpu.SemaphoreType.DMA,
            pltpu.SemaphoreType.DMA,
            pltpu.SemaphoreType.DMA,
            pltpu.SemaphoreType.DMA,
            pltpu.SemaphoreType.DMA,
        ],
    )(_body)
    return f(xt, geom, oiz)


_BP = 8192
_BO = 16384


def _repack_body(x_ref, o_ref):
    blk = x_ref[:, 0, :, :].reshape(32, _BP)
    o_ref[0] = jnp.transpose(blk, (1, 0))


def _outt_body(i_ref, o_ref):
    for k in range(_BO // 128):
        o_ref[:, k, :] = jnp.transpose(i_ref[pl.ds(k * 128, 128), :], (1, 0))


@jax.jit
def _repack(x):
    xv = x.reshape(2, 6, 16, L * L)
    out = pl.pallas_call(
        _repack_body,
        grid=(6, (L * L) // _BP),
        in_specs=[pl.BlockSpec((2, 1, 16, _BP), lambda i, j: (0, i, 0, j))],
        out_specs=pl.BlockSpec((1, _BP, 32), lambda i, j: (i, j, 0)),
        out_shape=jax.ShapeDtypeStruct((6, L * L, 32), jnp.float32),
    )(xv)
    return out.reshape(R, 32)


@jax.jit
def _outt(out_pm):
    out = pl.pallas_call(
        _outt_body,
        grid=(HW // _BO,),
        in_specs=[pl.BlockSpec((_BO, 32), lambda j: (j, 0))],
        out_specs=pl.BlockSpec((32, _BO // 128, 128), lambda j: (0, j, 0)),
        out_shape=jax.ShapeDtypeStruct((32, HW // 128, 128), jnp.float32),
    )(out_pm)
    return out.reshape(32, HW)


def kernel(x, XY0, idx0, XY1, idx1, XY2, idx2, XY3, idx3, XY4, idx4, XY5, idx5):
    XYs = [XY0, XY1, XY2, XY3, XY4, XY5]
    idxs = [idx0, idx1, idx2, idx3, idx4, idx5]
    equ = x.shape[0] // 6
    C = x.shape[1]
    xt = x.reshape(equ, 6, C, L * L).transpose(1, 3, 0, 2).reshape(R, equ * C)
    gx = jnp.concatenate([xy[:, 0] for xy in XYs])
    gy = jnp.concatenate([xy[:, 1] for xy in XYs])
    fb = jnp.concatenate([jnp.full((idxs[f].shape[0],), f * L * L, jnp.int32)
                          for f in range(6)])
    oi = jnp.concatenate(idxs)
    geom = jnp.stack([gx, gy, fb.astype(jnp.float32)])
    geom = geom.reshape(3, HW // S, S).transpose(1, 0, 2)
    geom = jnp.pad(geom, ((0, 4), (0, 0), (0, 0)))
    oiz = oi.reshape(HW // S, S)
    out_pm = _run(xt, geom, oiz)
    return out_pm.reshape(HW, equ, C).transpose(1, 2, 0).reshape(equ, C, 512, 1024)


# speedup vs baseline: 1.0581x; 1.0000x over previous
import functools

import jax
import jax.numpy as jnp
from jax import lax
from jax.experimental import pallas as pl
from jax.experimental.pallas import tpu as pltpu
from jax.experimental.pallas import tpu_sc as plsc

L = 256
HW = 512 * 1024
R = 6 * L * L
NW = 32
CHUNK = HW // NW
S = 128
NSUB = CHUNK // S
GP = 4


def _body(xt_h, geom_h, oiz_h, out_h,
          oi_v, geom_v, r_v, w_v, g_v, outA, outB,
          isemA, isemB, gsem0, gsem1, gsem2, gsem3, ssemA, ssemB):
    cid = lax.axis_index("c")
    sid = lax.axis_index("s")
    wid = sid * 2 + cid
    base = wid * CHUNK
    mbase = wid * NSUB

    ov = [outA, outB]
    isem = [isemA, isemB]
    gsem = [gsem0, gsem1, gsem2, gsem3]
    ssem = [ssemA, ssemB]

    def drain(dst, sem, src):
        pltpu.make_async_copy(src, dst, sem).wait()

    def addr(t, q4):
        def one(q, c2):
            cs = pl.ds(q * 16, 16)
            gxv = geom_v[q4, 0, cs]
            gyv = geom_v[q4, 1, cs]
            fbv = geom_v[q4, 2, cs].astype(jnp.int32)
            x0 = gxv.astype(jnp.int32)
            y0 = gyv.astype(jnp.int32)
            wx = gxv - x0.astype(jnp.float32)
            wy = gyv - y0.astype(jnp.float32)
            y1 = jnp.minimum(y0 + 1, L - 1)
            r0 = fbv + y0 * L + x0
            r1 = fbv + y1 * L + x0
            r_v[q4, 0, cs] = r0
            r_v[q4, 1, cs] = jnp.minimum(r0 + 1, R - 1)
            r_v[q4, 2, cs] = r1
            r_v[q4, 3, cs] = jnp.minimum(r1 + 1, R - 1)
            w_v[q4, 0, cs] = (1.0 - wx) * (1.0 - wy)
            w_v[q4, 1, cs] = wx * (1.0 - wy)
            w_v[q4, 2, cs] = (1.0 - wx) * wy
            w_v[q4, 3, cs] = wx * wy
            return c2
        lax.fori_loop(0, S // 16, one, 0)

    def fire_gathers(q4):
        for tap in range(4):
            pltpu.async_copy(xt_h.at[r_v.at[q4, tap]],
                             g_v.at[q4, tap], gsem[q4])

    def drain_gathers(q4):
        for tap in range(4):
            drain(g_v.at[q4, tap], gsem[q4], xt_h.at[pl.ds(0, S)])

    def combine(q4, p2):
        def one(g, c2):
            sl = pl.ds(g * 16, 16)
            wa = w_v[q4, 0, sl]
            wb = w_v[q4, 1, sl]
            wc = w_v[q4, 2, sl]
            wd = w_v[q4, 3, sl]
            lo = pl.ds(0, 16)
            hi = pl.ds(16, 16)
            for k in range(16):
                e = g * 16 + k
                ov[p2][e, lo] = (g_v[q4, 0, e, lo] * wa[k] + g_v[q4, 1, e, lo] * wb[k]
                                 + g_v[q4, 2, e, lo] * wc[k] + g_v[q4, 3, e, lo] * wd[k])
                ov[p2][e, hi] = (g_v[q4, 0, e, hi] * wa[k] + g_v[q4, 1, e, hi] * wb[k]
                                 + g_v[q4, 2, e, hi] * wc[k] + g_v[q4, 3, e, hi] * wd[k])
            return c2
        lax.fori_loop(0, S // 16, one, 0)

    def phase(t, q4, p2, drain_scatter):
        q4n2 = (q4 + 2) % 4
        drain(geom_v.at[q4n2], isem[p2], geom_h.at[0])
        addr(t + 2, q4n2)
        fire_gathers(q4n2)
        pltpu.async_copy(geom_h.at[mbase + t + 4], geom_v.at[q4], isem[p2])
        drain_gathers(q4)
        if drain_scatter:
            drain(ov[p2], ssem[p2], xt_h.at[pl.ds(0, S)])
        combine(q4, p2)
        pltpu.async_copy(ov[p2], out_h.at[oi_v.at[t]], ssem[p2])

    pltpu.sync_copy(oiz_h.at[pl.ds(wid * NSUB, NSUB)], oi_v)
    pltpu.sync_copy(geom_h.at[mbase], geom_v.at[0])
    pltpu.sync_copy(geom_h.at[mbase + 1], geom_v.at[1])
    addr(0, 0)
    addr(1, 1)
    fire_gathers(0)
    fire_gathers(1)
    pltpu.async_copy(geom_h.at[mbase + 2], geom_v.at[2], isem[0])
    pltpu.async_copy(geom_h.at[mbase + 3], geom_v.at[3], isem[1])
    phase(0, 0, 0, False)
    phase(1, 1, 1, False)
    phase(2, 2, 0, True)
    phase(3, 3, 1, True)

    def step(k, c2):
        t = 4 * k
        phase(t, 0,0, True)
        phase(t + 1, 1, 1, True)
        phase(t + 2, 2, 0, True)
        phase(t + 3, 3, 1, True)
        return c2
    lax.fori_loop(1, NSUB // 4, step, 0)

    drain_gathers(0)
    drain_gathers(1)
    drain(geom_v.at[2], isem[0], geom_h.at[0])
    drain(geom_v.at[3], isem[1], geom_h.at[0])
    drain(ov[0], ssem[0], xt_h.at[pl.ds(0, S)])
    drain(ov[1], ssem[1], xt_h.at[pl.ds(0, S)])


@jax.jit
def _run(xt, geom, oiz):
    mesh = plsc.VectorSubcoreMesh(core_axis_name="c", subcore_axis_name="s")
    f = functools.partial(
        pl.kernel,
        out_type=jax.ShapeDtypeStruct((HW, 32), jnp.float32),
        mesh=mesh,
        compiler_params=pltpu.CompilerParams(use_tc_tiling_on_sc=False),
        scratch_types=[
            pltpu.VMEM((NSUB, S), jnp.int32),
            pltpu.VMEM((GP, 3, S), jnp.float32),
            pltpu.VMEM((GP, 4, S), jnp.int32),
            pltpu.VMEM((GP, 4, S), jnp.float32),
            pltpu.VMEM((GP, 4, S, 32), jnp.float32),
            pltpu.VMEM((S, 32), jnp.float32),
            pltpu.VMEM((S, 32), jnp.float32),
            pltpu.SemaphoreType.DMA,
            pltpu.SemaphoreType.DMA,
            pltpu.SemaphoreType.DMA,
            pltpu.SemaphoreType.DMA,
            pltpu.SemaphoreType.DMA,
            pltpu.SemaphoreType.DMA,
            pltpu.SemaphoreType.DMA,
            pltpu.SemaphoreType.DMA,
        ],
    )(_body)
    return f(xt, geom, oiz)


def kernel(x, XY0, idx0, XY1, idx1, XY2, idx2, XY3, idx3, XY4, idx4, XY5, idx5):
    XYs = [XY0, XY1, XY2, XY3, XY4, XY5]
    idxs = [idx0, idx1, idx2, idx3, idx4, idx5]
    equ = x.shape[0] // 6
    C = x.shape[1]
    xt = x.reshape(equ, 6, C, L * L).transpose(1, 3, 0, 2).reshape(R, equ * C)
    gx = jnp.concatenate([xy[:, 0] for xy in XYs])
    gy = jnp.concatenate([xy[:, 1] for xy in XYs])
    fb = jnp.concatenate([jnp.full((idxs[f].shape[0],), f * L * L, jnp.int32)
                          for f in range(6)])
    oi = jnp.concatenate(idxs)
    geom = jnp.stack([gx, gy, fb.astype(jnp.float32)])
    geom = geom.reshape(3, HW // S, S).transpose(1, 0, 2)
    geom = jnp.pad(geom, ((0, 4), (0, 0), (0, 0)))
    oiz = oi.reshape(HW // S, S)
    out_pm = _run(xt, geom, oiz)
    return out_pm.reshape(HW, equ, C).transpose(1, 2, 0).reshape(equ, C, 512, 1024)
